# baseline (speedup 1.0000x reference)
import functools

import jax
import jax.numpy as jnp
from jax import lax
from jax.experimental import pallas as pl
from jax.experimental.pallas import tpu as pltpu
from jax.experimental.pallas import tpu_sc as plsc

_N_NODES = 100000
_D = 128
_B = 16384
_EPS = 1.0 / (float(_N_NODES) * float(_N_NODES))

_NC, _NS, _L = 2, 16, 16
_NW = _NC * _NS
_W = _B // _NW
_C = 128
_NCH = _W // _C

_R = 5000
_NB = _N_NODES // _R


def _sc_body(dep, head, rel, tail, Hr, Hi, Tr, Ti, Rr, Ri, out,
             hidx, tidx, relv, ure, uim, vre, vim, sv, rrv, riv, sem):
    del dep
    wid = lax.axis_index("s") * _NC + lax.axis_index("c")
    pltpu.sync_copy(Rr, rrv)
    pltpu.sync_copy(Ri, riv)
    nch16 = _D // _L
    r0r = [rrv[0, pl.ds(c * _L, _L)] for c in range(nch16)]
    r0i = [riv[0, pl.ds(c * _L, _L)] for c in range(nch16)]
    r1r = [rrv[1, pl.ds(c * _L, _L)] for c in range(nch16)]
    r1i = [riv[1, pl.ds(c * _L, _L)] for c in range(nch16)]
    zero = jnp.zeros((_L,), jnp.float32)
    iota = lax.iota(jnp.int32, _L)

    def chunk(ci, carry):
        base = pl.multiple_of(wid * _W + ci * _C, _C)
        pltpu.sync_copy(head.at[pl.ds(base, _C)], hidx)
        pltpu.sync_copy(tail.at[pl.ds(base, _C)], tidx)
        pltpu.sync_copy(rel.at[pl.ds(base, _C)], relv)
        c1 = pltpu.async_copy(Hr.at[hidx], ure, sem)
        c2 = pltpu.async_copy(Hi.at[hidx], uim, sem)
        c3 = pltpu.async_copy(Tr.at[tidx], vre, sem)
        c4 = pltpu.async_copy(Ti.at[tidx], vim, sem)
        c1.wait()
        c2.wait()
        c3.wait()
        c4.wait()

        def rbody(row, carry2):
            g0, g1 = carry2
            acc0 = zero
            acc1 = zero
            for c in range(nch16):
                a = ure[row, pl.ds(c * _L, _L)]
                b = uim[row, pl.ds(c * _L, _L)]
                f = vre[row, pl.ds(c * _L, _L)]
                e = vim[row, pl.ds(c * _L, _L)]
                pp = a * f + b * e
                qq = a * e - b * f
                acc0 = acc0 + pp * r0r[c] + qq * r0i[c]
                acc1 = acc1 + pp * r1r[c] + qq * r1i[c]
            lane = row % _L
            g0 = jnp.where(iota == lane, jnp.full((_L,), jnp.sum(acc0)), g0)
            g1 = jnp.where(iota == lane, jnp.full((_L,), jnp.sum(acc1)), g1)

            @pl.when(lane == _L - 1)
            def _flush():
                gbase = pl.multiple_of(row - (_L - 1), _L)
                rl = relv[pl.ds(gbase, _L)]
                sv[pl.ds(gbase, _L)] = jnp.where(rl == 0, g0, g1)

            done = lane == _L - 1
            g0 = jnp.where(done, zero, g0)
            g1 = jnp.where(done, zero, g1)
            return (g0, g1)

        lax.fori_loop(0, _C, rbody, (zero, zero), unroll=2)
        pltpu.sync_copy(sv, out.at[pl.ds(base, _C)])
        return carry

    lax.fori_loop(0, _NCH, chunk, 0)


@functools.lru_cache(maxsize=1)
def _sc_score_fn():
    return pl.kernel(
        _sc_body,
        out_type=jax.ShapeDtypeStruct((_B,), jnp.float32),
        mesh=plsc.VectorSubcoreMesh(core_axis_name="c", subcore_axis_name="s"),
        compiler_params=pltpu.CompilerParams(needs_layout_passes=False),
        scratch_types=[
            pltpu.VMEM((_C,), jnp.int32),
            pltpu.VMEM((_C,), jnp.int32),
            pltpu.VMEM((_C,), jnp.int32),
            pltpu.VMEM((_C, _D), jnp.float32),
            pltpu.VMEM((_C, _D), jnp.float32),
            pltpu.VMEM((_C, _D), jnp.float32),
            pltpu.VMEM((_C, _D), jnp.float32),
            pltpu.VMEM((_C,), jnp.float32),
            pltpu.VMEM((2, _D), jnp.float32),
            pltpu.VMEM((2, _D), jnp.float32),
            pltpu.SemaphoreType.DMA,
        ],
    )


_K1 = 2


def _gram_acc(hr_ref, hi_ref, tr_ref, ti_ref, acca, accb, first):
    @pl.when(first)
    def _init():
        acca[...] = jnp.zeros_like(acca)
        accb[...] = jnp.zeros_like(accb)

    a = jnp.concatenate([hr_ref[...], hi_ref[...]], axis=1).astype(jnp.bfloat16)
    b = jnp.concatenate([tr_ref[...], ti_ref[...]], axis=1).astype(jnp.bfloat16)

    def gram(x):
        return lax.dot_general(x, x, (((0,), (0,)), ((), ())),
                               preferred_element_type=jnp.float32)

    acca[...] += gram(a)
    accb[...] += gram(b)


def _gram1_body(hr_ref, hi_ref, tr_ref, ti_ref, outa_ref, outb_ref,
                acca, accb):
    i = pl.program_id(0)
    _gram_acc(hr_ref, hi_ref, tr_ref, ti_ref, acca, accb, i == 0)

    @pl.when(i == _K1 - 1)
    def _emit():
        outa_ref[...] = acca[...]
        outb_ref[...] = accb[...]


def _gram2_body(hr_ref, hi_ref, tr_ref, ti_ref, ina_ref, inb_ref,
                rr_ref, ri_ref, z_ref, acca, accb):
    i = pl.program_id(0)

    @pl.when(i == 0)
    def _init():
        acca[...] = ina_ref[...]
        accb[...] = inb_ref[...]

    _gram_acc(hr_ref, hi_ref, tr_ref, ti_ref, acca, accb, i < 0)

    @pl.when(i == _NB - _K1 - 1)
    def _finish():
        ga = acca[...]
        gb = accb[...]
        g1 = ga[:_D, :_D]
        g3 = ga[:_D, _D:]
        g3t = ga[_D:, :_D]
        g2 = ga[_D:, _D:]
        g4 = gb[:_D, :_D]
        g6 = gb[:_D, _D:]
        g5 = gb[_D:, _D:]
        m_rr = g1 * g4 + g2 * g5 + 2.0 * g3 * g6
        m_ii = g1 * g5 + g2 * g4 - 2.0 * g3t * g6
        m_ri = 2.0 * (g1 * g6 - g3 * g4)
        m_ir = 2.0 * (g3 * g5 - g2 * g6)
        rows = []
        for r in range(2):
            pr = rr_ref[r, :]
            pi = ri_ref[r, :]
            w_rr = pr[:, None] * pr[None, :]
            w_ii = pi[:, None] * pi[None, :]
            w_ri = pr[:, None] * pi[None, :]
            w_ir = pi[:, None] * pr[None, :]
            z = jnp.sum(w_rr * m_rr + w_ii * m_ii + w_ri * m_ri + w_ir * m_ir)
            rows.append(jnp.full((1, _D), jnp.log1p(z), jnp.float32))
        z_ref[...] = jnp.concatenate(rows, axis=0)


def _gram_z1(Hr, Hi, Tr, Ti):
    tab = pl.BlockSpec((_R, _D), lambda i: (i, 0))
    acc = jax.ShapeDtypeStruct((2 * _D, 2 * _D), jnp.float32)
    return pl.pallas_call(
        _gram1_body,
        grid=(_K1,),
        in_specs=[tab, tab, tab, tab],
        out_specs=[pl.BlockSpec((2 * _D, 2 * _D), lambda i: (0, 0))] * 2,
        out_shape=[acc, acc],
        scratch_shapes=[pltpu.VMEM((2 * _D, 2 * _D), jnp.float32),
                        pltpu.VMEM((2 * _D, 2 * _D), jnp.float32)],
    )(Hr, Hi, Tr, Ti)


def _gram_z2(Hr, Hi, Tr, Ti, acca, accb, Rr, Ri):
    tab = pl.BlockSpec((_R, _D), lambda i: (i + _K1, 0))
    accs = pl.BlockSpec((2 * _D, 2 * _D), lambda i: (0, 0))
    rel = pl.BlockSpec((2, _D), lambda i: (0, 0))
    return pl.pallas_call(
        _gram2_body,
        grid=(_NB - _K1,),
        in_specs=[tab, tab, tab, tab, accs, accs, rel, rel],
        out_specs=pl.BlockSpec((2, _D), lambda i: (0, 0)),
        out_shape=jax.ShapeDtypeStruct((2, _D), jnp.float32),
        scratch_shapes=[pltpu.VMEM((2 * _D, 2 * _D), jnp.float32),
                        pltpu.VMEM((2 * _D, 2 * _D), jnp.float32)],
    )(Hr, Hi, Tr, Ti, acca, accb, Rr, Ri)


def _combine_body(s_ref, rel_ref, z_ref, out_ref):
    s = s_ref[...]
    rel = rel_ref[...]
    lz = jnp.where(rel == 0, z_ref[0, 0], z_ref[1, 0])
    out_ref[...] = jnp.log(s * s + _EPS) - lz


def _combine(s2d, rel2d, z):
    return pl.pallas_call(
        _combine_body,
        out_shape=jax.ShapeDtypeStruct((_D, _B // _D), jnp.float32),
    )(s2d, rel2d, z)


def kernel(head, relation, tail, Hr, Hi, Tr, Ti, Rr, Ri):
    head = head.astype(jnp.int32)
    relation = relation.astype(jnp.int32)
    tail = tail.astype(jnp.int32)
    acca, accb = _gram_z1(Hr, Hi, Tr, Ti)
    s = _sc_score_fn()(acca, head, relation, tail, Hr, Hi, Tr, Ti, Rr, Ri)
    z = _gram_z2(Hr, Hi, Tr, Ti, acca, accb, Rr, Ri)
    out = _combine(s.reshape(_D, _B // _D), relation.reshape(_D, _B // _D), z)
    return out.reshape(_B)


# speedup vs baseline: 1.0418x; 1.0418x over previous
import functools

import jax
import jax.numpy as jnp
from jax import lax
from jax.experimental import pallas as pl
from jax.experimental.pallas import tpu as pltpu
from jax.experimental.pallas import tpu_sc as plsc

_N_NODES = 100000
_D = 128
_B = 16384
_EPS = 1.0 / (float(_N_NODES) * float(_N_NODES))

_NC, _NS, _L = 2, 16, 16
_NW = _NC * _NS
_W = _B // _NW
_C = 128
_NCH = _W // _C

_R = 5000
_NB = _N_NODES // _R


def _sc_body(head, rel, tail, Hr, Hi, Tr, Ti, Rr, Ri, out,
             hidx, tidx, relv, ure, uim, vre, vim, sv, rrv, riv, sem):
    wid = lax.axis_index("s") * _NC + lax.axis_index("c")
    pltpu.sync_copy(Rr, rrv)
    pltpu.sync_copy(Ri, riv)
    nch16 = _D // _L
    r0r = [rrv[0, pl.ds(c * _L, _L)] for c in range(nch16)]
    r0i = [riv[0, pl.ds(c * _L, _L)] for c in range(nch16)]
    r1r = [rrv[1, pl.ds(c * _L, _L)] for c in range(nch16)]
    r1i = [riv[1, pl.ds(c * _L, _L)] for c in range(nch16)]
    zero = jnp.zeros((_L,), jnp.float32)
    iota = lax.iota(jnp.int32, _L)

    def chunk(ci, carry):
        base = pl.multiple_of(wid * _W + ci * _C, _C)
        pltpu.sync_copy(head.at[pl.ds(base, _C)], hidx)
        pltpu.sync_copy(tail.at[pl.ds(base, _C)], tidx)
        pltpu.sync_copy(rel.at[pl.ds(base, _C)], relv)
        c1 = pltpu.async_copy(Hr.at[hidx], ure, sem)
        c2 = pltpu.async_copy(Hi.at[hidx], uim, sem)
        c3 = pltpu.async_copy(Tr.at[tidx], vre, sem)
        c4 = pltpu.async_copy(Ti.at[tidx], vim, sem)
        c1.wait()
        c2.wait()
        c3.wait()
        c4.wait()

        def rbody(row, carry2):
            g0, g1 = carry2
            acc0 = zero
            acc1 = zero
            for c in range(nch16):
                a = ure[row, pl.ds(c * _L, _L)]
                b = uim[row, pl.ds(c * _L, _L)]
                f = vre[row, pl.ds(c * _L, _L)]
                e = vim[row, pl.ds(c * _L, _L)]
                pp = a * f + b * e
                qq = a * e - b * f
                acc0 = acc0 + pp * r0r[c] + qq * r0i[c]
                acc1 = acc1 + pp * r1r[c] + qq * r1i[c]
            lane = row % _L
            g0 = jnp.where(iota == lane, jnp.full((_L,), jnp.sum(acc0)), g0)
            g1 = jnp.where(iota == lane, jnp.full((_L,), jnp.sum(acc1)), g1)

            @pl.when(lane == _L - 1)
            def _flush():
                gbase = pl.multiple_of(row - (_L - 1), _L)
                rl = relv[pl.ds(gbase, _L)]
                sv[pl.ds(gbase, _L)] = jnp.where(rl == 0, g0, g1)

            done = lane == _L - 1
            g0 = jnp.where(done, zero, g0)
            g1 = jnp.where(done, zero, g1)
            return (g0, g1)

        lax.fori_loop(0, _C, rbody, (zero, zero), unroll=2)
        pltpu.sync_copy(sv, out.at[pl.ds(base, _C)])
        return carry

    lax.fori_loop(0, _NCH, chunk, 0)


@functools.lru_cache(maxsize=1)
def _sc_score_fn():
    return pl.kernel(
        _sc_body,
        out_type=jax.ShapeDtypeStruct((_B,), jnp.float32),
        mesh=plsc.VectorSubcoreMesh(core_axis_name="c", subcore_axis_name="s"),
        compiler_params=pltpu.CompilerParams(needs_layout_passes=False),
        scratch_types=[
            pltpu.VMEM((_C,), jnp.int32),
            pltpu.VMEM((_C,), jnp.int32),
            pltpu.VMEM((_C,), jnp.int32),
            pltpu.VMEM((_C, _D), jnp.float32),
            pltpu.VMEM((_C, _D), jnp.float32),
            pltpu.VMEM((_C, _D), jnp.float32),
            pltpu.VMEM((_C, _D), jnp.float32),
            pltpu.VMEM((_C,), jnp.float32),
            pltpu.VMEM((2, _D), jnp.float32),
            pltpu.VMEM((2, _D), jnp.float32),
            pltpu.SemaphoreType.DMA,
        ],
    )


def _gram_body(hr_ref, hi_ref, tr_ref, ti_ref, rr_ref, ri_ref, z_ref,
               acca, accb):
    i = pl.program_id(0)

    @pl.when(i == 0)
    def _init():
        acca[...] = jnp.zeros_like(acca)
        accb[...] = jnp.zeros_like(accb)

    a = jnp.concatenate([hr_ref[...], hi_ref[...]], axis=1).astype(jnp.bfloat16)
    b = jnp.concatenate([tr_ref[...], ti_ref[...]], axis=1).astype(jnp.bfloat16)

    def gram(x):
        return lax.dot_general(x, x, (((0,), (0,)), ((), ())),
                               preferred_element_type=jnp.float32)

    acca[...] += gram(a)
    accb[...] += gram(b)

    @pl.when(i == _NB - 1)
    def _finish():
        ga = acca[...]
        gb = accb[...]
        g1 = ga[:_D, :_D]
        g3 = ga[:_D, _D:]
        g3t = ga[_D:, :_D]
        g2 = ga[_D:, _D:]
        g4 = gb[:_D, :_D]
        g6 = gb[:_D, _D:]
        g5 = gb[_D:, _D:]
        m_rr = g1 * g4 + g2 * g5 + 2.0 * g3 * g6
        m_ii = g1 * g5 + g2 * g4 - 2.0 * g3t * g6
        m_ri = 2.0 * (g1 * g6 - g3 * g4)
        m_ir = 2.0 * (g3 * g5 - g2 * g6)
        rows = []
        for r in range(2):
            pr = rr_ref[r, :]
            pi = ri_ref[r, :]
            w_rr = pr[:, None] * pr[None, :]
            w_ii = pi[:, None] * pi[None, :]
            w_ri = pr[:, None] * pi[None, :]
            w_ir = pi[:, None] * pr[None, :]
            z = jnp.sum(w_rr * m_rr + w_ii * m_ii + w_ri * m_ri + w_ir * m_ir)
            rows.append(jnp.full((1, _D), jnp.log1p(z), jnp.float32))
        z_ref[...] = jnp.concatenate(rows, axis=0)


def _gram_z(Hr, Hi, Tr, Ti, Rr, Ri):
    tab = pl.BlockSpec((_R, _D), lambda i: (i, 0))
    rel = pl.BlockSpec((2, _D), lambda i: (0, 0))
    return pl.pallas_call(
        _gram_body,
        grid=(_NB,),
        in_specs=[tab, tab, tab, tab, rel, rel],
        out_specs=pl.BlockSpec((2, _D), lambda i: (0, 0)),
        out_shape=jax.ShapeDtypeStruct((2, _D), jnp.float32),
        scratch_shapes=[pltpu.VMEM((2 * _D, 2 * _D), jnp.float32),
                        pltpu.VMEM((2 * _D, 2 * _D), jnp.float32)],
    )(Hr, Hi, Tr, Ti, Rr, Ri)


def _combine_body(s_ref, rel_ref, z_ref, out_ref):
    s = s_ref[...]
    rel = rel_ref[...]
    lz = jnp.where(rel == 0, z_ref[0, 0], z_ref[1, 0])
    out_ref[...] = jnp.log(s * s + _EPS) - lz


def _combine(s2d, rel2d, z):
    return pl.pallas_call(
        _combine_body,
        out_shape=jax.ShapeDtypeStruct((_D, _B // _D), jnp.float32),
    )(s2d, rel2d, z)


def kernel(head, relation, tail, Hr, Hi, Tr, Ti, Rr, Ri):
    head = head.astype(jnp.int32)
    relation = relation.astype(jnp.int32)
    tail = tail.astype(jnp.int32)
    s = _sc_score_fn()(head, relation, tail, Hr, Hi, Tr, Ti, Rr, Ri)
    z = _gram_z(Hr, Hi, Tr, Ti, Rr, Ri)
    out = _combine(s.reshape(_D, _B // _D), relation.reshape(_D, _B // _D), z)
    return out.reshape(_B)
